# initial kernel scaffold (unmeasured)
import jax
import jax.numpy as jnp
from jax import lax
from jax.experimental import pallas as pl
from jax.experimental.pallas import tpu as pltpu

B = 16
H = 16
D = 64
SCALE = D ** -0.5


def kernel(Q, K, V):
    b, _, h, d = Q.shape
    kv = K.shape[1]

    Q2 = Q.reshape(b, h, d)

    def body(q_ref, k_ref, v_ref, out_ref,
             o_send, o_recv, l_send, l_recv, send_sems, recv_sems):
        bi = pl.program_id(0)
        nb = pl.num_programs(0)

        q = q_ref[0]
        k = k_ref[0]
        v = v_ref[0]

        s = (k * q[None, :, :]).sum(axis=-1) * SCALE
        p = jnp.exp(s)
        l_b = p.sum(axis=0, keepdims=True)
        o_b = (p[:, :, None] * v).sum(axis=0)

        l_send[pl.ds(bi, 1), :] = l_b
        o_send[pl.ds(bi, 1), :, :] = o_b[None]

        @pl.when(bi == nb - 1)
        def _():
            my_x = lax.axis_index("x")
            my_y = lax.axis_index("y")
            peer = (1 - my_x, my_y)
            rdma_o = pltpu.make_async_remote_copy(
                src_ref=o_send, dst_ref=o_recv,
                send_sem=send_sems.at[0], recv_sem=recv_sems.at[0],
                device_id=peer, device_id_type=pl.DeviceIdType.MESH,
            )
            rdma_l = pltpu.make_async_remote_copy(
                src_ref=l_send, dst_ref=l_recv,
                send_sem=send_sems.at[1], recv_sem=recv_sems.at[1],
                device_id=peer, device_id_type=pl.DeviceIdType.MESH,
            )
            rdma_o.start()
            rdma_l.start()
            rdma_o.wait()
            rdma_l.wait()

            l_tot = l_send[...] + l_recv[...]
            o_tot = o_send[...] + o_recv[...]
            out_ref[...] = o_tot / l_tot[:, :, None]

    out = pl.pallas_call(
        body,
        grid=(b,),
        in_specs=[
            pl.BlockSpec((1, h, d), lambda i: (i, 0, 0)),
            pl.BlockSpec((1, kv, h, d), lambda i: (i, 0, 0, 0)),
            pl.BlockSpec((1, kv, h, d), lambda i: (i, 0, 0, 0)),
        ],
        out_specs=pl.BlockSpec((b, h, d), lambda i: (0, 0, 0)),
        out_shape=jax.ShapeDtypeStruct((b, h, d), jnp.float32),
        scratch_shapes=[
            pltpu.VMEM((b, h, d), jnp.float32),
            pltpu.VMEM((b, h, d), jnp.float32),
            pltpu.VMEM((b, h), jnp.float32),
            pltpu.VMEM((b, h), jnp.float32),
            pltpu.SemaphoreType.DMA((2,)),
            pltpu.SemaphoreType.DMA((2,)),
        ],
        compiler_params=pltpu.CompilerParams(collective_id=0),
    )(Q2, K, V)

    return out.reshape(b, 1, h, d)


# baseline (device time: 305549 ns/iter reference)
import jax
import jax.numpy as jnp
from jax import lax
from jax.experimental import pallas as pl
from jax.experimental.pallas import tpu as pltpu

B = 16
H = 16
D = 64
SCALE = D ** -0.5


def kernel(Q, K, V):
    b, _, h, d = Q.shape
    kv = K.shape[1]

    Q2 = Q.reshape(b, h, d)

    def body(q_ref, k_ref, v_ref, out_ref,
             o_send, o_recv, l_send, l_recv, send_sems, recv_sems):
        bi = pl.program_id(0)
        nb = pl.num_programs(0)

        q = q_ref[0]
        k = k_ref[0]
        v = v_ref[0]

        s = (k * q[None, :, :]).sum(axis=-1) * SCALE
        p = jnp.exp(s)
        l_b = p.sum(axis=0, keepdims=True)
        o_b = (p[:, :, None] * v).sum(axis=0)

        l_send[pl.ds(bi, 1), :] = l_b
        o_send[pl.ds(bi, 1), :, :] = o_b[None]

        @pl.when(bi == nb - 1)
        def _():
            my_x = lax.axis_index("x")
            my_y = lax.axis_index("y")
            peer = (1 - my_x, my_y)
            rdma_o = pltpu.make_async_remote_copy(
                src_ref=o_send, dst_ref=o_recv,
                send_sem=send_sems.at[0], recv_sem=recv_sems.at[0],
                device_id=peer, device_id_type=pl.DeviceIdType.MESH,
            )
            rdma_l = pltpu.make_async_remote_copy(
                src_ref=l_send, dst_ref=l_recv,
                send_sem=send_sems.at[1], recv_sem=recv_sems.at[1],
                device_id=peer, device_id_type=pl.DeviceIdType.MESH,
            )
            rdma_o.start()
            rdma_l.start()
            rdma_o.wait()
            rdma_l.wait()

            l_tot = l_send[...] + l_recv[...]
            o_tot = o_send[...] + o_recv[...]
            out_ref[...] = o_tot / l_tot[:, :, None]

    out = pl.pallas_call(
        body,
        grid=(b,),
        in_specs=[
            pl.BlockSpec((1, h, d), lambda i: (i, 0, 0)),
            pl.BlockSpec((1, kv, h, d), lambda i: (i, 0, 0, 0)),
            pl.BlockSpec((1, kv, h, d), lambda i: (i, 0, 0, 0)),
        ],
        out_specs=pl.BlockSpec((b, h, d), lambda i: (0, 0, 0)),
        out_shape=jax.ShapeDtypeStruct((b, h, d), jnp.float32),
        scratch_shapes=[
            pltpu.VMEM((b, h, d), jnp.float32),
            pltpu.VMEM((b, h, d), jnp.float32),
            pltpu.VMEM((b, h), jnp.float32),
            pltpu.VMEM((b, h), jnp.float32),
            pltpu.SemaphoreType.DMA((2,)),
            pltpu.SemaphoreType.DMA((2,)),
        ],
        compiler_params=pltpu.CompilerParams(
            vmem_limit_bytes=100 * 1024 * 1024,
        ),
    )(Q2, K, V)

    return out.reshape(b, 1, h, d)


# device time: 174284 ns/iter; 1.7532x vs baseline; 1.7532x over previous
import jax
import jax.numpy as jnp
from jax import lax
from jax.experimental import pallas as pl
from jax.experimental.pallas import tpu as pltpu

SCALE = 64 ** -0.5


def kernel(Q, K, V):
    b, _, h, d = Q.shape
    kv = K.shape[1]
    hd = h * d
    kv_half = kv // 2

    Kf = K.reshape(b, kv, hd)
    Vf = V.reshape(b, kv, hd)
    eye = jnp.eye(h, dtype=jnp.float32)
    Qr = Q.reshape(b, h, d)
    QM = (Qr[:, :, :, None] * eye[None, :, None, :]).reshape(b, hd, h).astype(jnp.bfloat16)

    def body(y_ref, qm_ref, k_ref, v_ref, out_ref,
             o_send, o_recv1, o_mid, o_recv2,
             l_send, l_recv1, l_mid, l_recv2,
             send_sems, recv_sems):
        bi = pl.program_id(0)
        nb = pl.num_programs(0)

        qm = qm_ref[0]
        kf = k_ref[0].astype(jnp.bfloat16)
        vf = v_ref[0].astype(jnp.bfloat16)

        s = jax.lax.dot(kf, qm, preferred_element_type=jnp.float32)
        p = jnp.exp(s * SCALE)
        l_b = p.sum(axis=0, keepdims=True)
        pt = p.T.astype(jnp.bfloat16)
        o_full = jax.lax.dot(pt, vf, preferred_element_type=jnp.float32)

        sub = lax.broadcasted_iota(jnp.int32, (h, hd), 0)
        lane = lax.broadcasted_iota(jnp.int32, (h, hd), 1)
        mask = (sub == lane // d)
        o_flat = jnp.where(mask, o_full, 0.0).sum(axis=0, keepdims=True)

        l_send[pl.ds(bi, 1), :] = l_b
        o_send[pl.ds(bi, 1), :] = o_flat

        @pl.when(bi == nb - 1)
        def _():
            my_x = lax.axis_index("x")
            my_y = lax.axis_index("y")

            peer_y = (my_x, 1 - my_y)
            r1o = pltpu.make_async_remote_copy(
                src_ref=o_send, dst_ref=o_recv1,
                send_sem=send_sems.at[0], recv_sem=recv_sems.at[0],
                device_id=peer_y, device_id_type=pl.DeviceIdType.MESH,
            )
            r1l = pltpu.make_async_remote_copy(
                src_ref=l_send, dst_ref=l_recv1,
                send_sem=send_sems.at[1], recv_sem=recv_sems.at[1],
                device_id=peer_y, device_id_type=pl.DeviceIdType.MESH,
            )
            r1o.start()
            r1l.start()
            r1o.wait()
            r1l.wait()
            o_mid[...] = o_send[...] + o_recv1[...]
            l_mid[...] = l_send[...] + l_recv1[...]

            peer_x = (1 - my_x, my_y)
            r2o = pltpu.make_async_remote_copy(
                src_ref=o_mid, dst_ref=o_recv2,
                send_sem=send_sems.at[2], recv_sem=recv_sems.at[2],
                device_id=peer_x, device_id_type=pl.DeviceIdType.MESH,
            )
            r2l = pltpu.make_async_remote_copy(
                src_ref=l_mid, dst_ref=l_recv2,
                send_sem=send_sems.at[3], recv_sem=recv_sems.at[3],
                device_id=peer_x, device_id_type=pl.DeviceIdType.MESH,
            )
            r2o.start()
            r2l.start()
            r2o.wait()
            r2l.wait()

            o_tot = o_recv2[...] + o_mid[...]
            l_tot = l_recv2[...] + l_mid[...]
            sub2 = lax.broadcasted_iota(jnp.int32, (h, hd), 0)
            lane2 = lax.broadcasted_iota(jnp.int32, (h, hd), 1)
            e = (sub2 == lane2 // d).astype(jnp.float32)
            l2 = jax.lax.dot(l_tot, e, preferred_element_type=jnp.float32)
            out_ref[...] = o_tot / l2

    grid_spec = pltpu.PrefetchScalarGridSpec(
        num_scalar_prefetch=1,
        grid=(b,),
        in_specs=[
            pl.BlockSpec((1, hd, h), lambda i, y_ref: (i, 0, 0)),
            pl.BlockSpec((1, kv_half, hd), lambda i, y_ref: (i, y_ref[0], 0)),
            pl.BlockSpec((1, kv_half, hd), lambda i, y_ref: (i, y_ref[0], 0)),
        ],
        out_specs=pl.BlockSpec((b, hd), lambda i, y_ref: (0, 0)),
        scratch_shapes=[
            pltpu.VMEM((b, hd), jnp.float32),
            pltpu.VMEM((b, hd), jnp.float32),
            pltpu.VMEM((b, hd), jnp.float32),
            pltpu.VMEM((b, hd), jnp.float32),
            pltpu.VMEM((b, h), jnp.float32),
            pltpu.VMEM((b, h), jnp.float32),
            pltpu.VMEM((b, h), jnp.float32),
            pltpu.VMEM((b, h), jnp.float32),
            pltpu.SemaphoreType.DMA((4,)),
            pltpu.SemaphoreType.DMA((4,)),
        ],
    )

    y_idx = jnp.reshape(lax.axis_index("y"), (1,)).astype(jnp.int32)

    out = pl.pallas_call(
        body,
        grid_spec=grid_spec,
        out_shape=jax.ShapeDtypeStruct((b, hd), jnp.float32),
        compiler_params=pltpu.CompilerParams(
            vmem_limit_bytes=100 * 1024 * 1024,
        ),
    )(y_idx, QM, Kf, Vf)

    return out.reshape(b, 1, h, d)


# device time: 37896 ns/iter; 8.0628x vs baseline; 4.5990x over previous
import jax
import jax.numpy as jnp
from jax import lax
from jax.experimental import pallas as pl
from jax.experimental.pallas import tpu as pltpu

SCALE = 64 ** -0.5
NSLOT = 4


def kernel(Q, K, V):
    b, _, h, d = Q.shape
    kv = K.shape[1]
    bh = b // 2

    Qr = Q.reshape(b, h, d)
    Kt = jnp.transpose(K, (0, 2, 3, 1))
    Vt = jnp.transpose(V, (0, 2, 3, 1))

    def body(q_ref, k_hbm, v_hbm, out_ref,
             kbuf, vbuf, o_half, o_oth, o_mid, o_recv2,
             l_half, l_oth, l_mid, l_recv2,
             ksems, vsems, send_sems, recv_sems):
        my_x = lax.axis_index("x")
        my_y = lax.axis_index("y")
        b0 = my_y * bh

        hh = h // 2

        def k_copy(bi, slot, half):
            sl = pl.ds(half * hh, hh)
            return pltpu.make_async_copy(
                k_hbm.at[b0 + bi, sl], kbuf.at[slot, sl], ksems.at[slot, half])

        def v_copy(bi, slot, half):
            sl = pl.ds(half * hh, hh)
            return pltpu.make_async_copy(
                v_hbm.at[b0 + bi, sl], vbuf.at[slot, sl], vsems.at[slot, half])

        for bi in range(NSLOT):
            for half in range(2):
                k_copy(bi, bi % NSLOT, half).start()
                v_copy(bi, bi % NSLOT, half).start()

        for bi in range(bh):
            slot = bi % NSLOT
            for half in range(2):
                k_copy(bi, slot, half).wait()
                v_copy(bi, slot, half).wait()

            q3 = q_ref[pl.ds(b0 + bi, 1)][0][:, None, :].astype(jnp.bfloat16)
            kt = kbuf[slot].astype(jnp.bfloat16)
            vt = vbuf[slot].astype(jnp.bfloat16)

            s3 = lax.dot_general(
                q3, kt,
                dimension_numbers=(((2,), (1,)), ((0,), (0,))),
                preferred_element_type=jnp.float32,
            )
            p3 = jnp.exp(s3 * SCALE)
            l3 = p3.sum(axis=2)
            o3 = lax.dot_general(
                vt, p3.astype(jnp.bfloat16),
                dimension_numbers=(((2,), (2,)), ((0,), (0,))),
                preferred_element_type=jnp.float32,
            )

            o_half[pl.ds(bi, 1), :, :] = o3[:, :, 0][None]
            l_half[pl.ds(bi, 1), :] = l3[:, 0][None]

            if bi + NSLOT < bh:
                for half in range(2):
                    k_copy(bi + NSLOT, slot, half).start()
                    v_copy(bi + NSLOT, slot, half).start()

        peer_y = (my_x, 1 - my_y)
        r1o = pltpu.make_async_remote_copy(
            src_ref=o_half, dst_ref=o_oth,
            send_sem=send_sems.at[0], recv_sem=recv_sems.at[0],
            device_id=peer_y, device_id_type=pl.DeviceIdType.MESH,
        )
        r1l = pltpu.make_async_remote_copy(
            src_ref=l_half, dst_ref=l_oth,
            send_sem=send_sems.at[1], recv_sem=recv_sems.at[1],
            device_id=peer_y, device_id_type=pl.DeviceIdType.MESH,
        )
        r1o.start()
        r1l.start()
        r1o.wait()
        r1l.wait()
        o_mid[pl.ds(b0, bh), :, :] = o_half[...]
        l_mid[pl.ds(b0, bh), :] = l_half[...]
        ob0 = (1 - my_y) * bh
        o_mid[pl.ds(ob0, bh), :, :] = o_oth[...]
        l_mid[pl.ds(ob0, bh), :] = l_oth[...]

        peer_x = (1 - my_x, my_y)
        r2o = pltpu.make_async_remote_copy(
            src_ref=o_mid, dst_ref=o_recv2,
            send_sem=send_sems.at[2], recv_sem=recv_sems.at[2],
            device_id=peer_x, device_id_type=pl.DeviceIdType.MESH,
        )
        r2l = pltpu.make_async_remote_copy(
            src_ref=l_mid, dst_ref=l_recv2,
            send_sem=send_sems.at[3], recv_sem=recv_sems.at[3],
            device_id=peer_x, device_id_type=pl.DeviceIdType.MESH,
        )
        r2o.start()
        r2l.start()
        r2o.wait()
        r2l.wait()

        o_tot = o_recv2[...] + o_mid[...]
        l_tot = l_recv2[...] + l_mid[...]
        out_ref[...] = (o_tot / l_tot[:, :, None])[:, None]

    out = pl.pallas_call(
        body,
        in_specs=[
            pl.BlockSpec(memory_space=pltpu.MemorySpace.VMEM),
            pl.BlockSpec(memory_space=pl.ANY),
            pl.BlockSpec(memory_space=pl.ANY),
        ],
        out_specs=pl.BlockSpec(memory_space=pltpu.MemorySpace.VMEM),
        out_shape=jax.ShapeDtypeStruct((b, 1, h, d), jnp.float32),
        scratch_shapes=[
            pltpu.VMEM((NSLOT, h, d, kv), jnp.float32),
            pltpu.VMEM((NSLOT, h, d, kv), jnp.float32),
            pltpu.VMEM((bh, h, d), jnp.float32),
            pltpu.VMEM((bh, h, d), jnp.float32),
            pltpu.VMEM((b, h, d), jnp.float32),
            pltpu.VMEM((b, h, d), jnp.float32),
            pltpu.VMEM((bh, h), jnp.float32),
            pltpu.VMEM((bh, h), jnp.float32),
            pltpu.VMEM((b, h), jnp.float32),
            pltpu.VMEM((b, h), jnp.float32),
            pltpu.SemaphoreType.DMA((NSLOT, 2)),
            pltpu.SemaphoreType.DMA((NSLOT, 2)),
            pltpu.SemaphoreType.DMA((4,)),
            pltpu.SemaphoreType.DMA((4,)),
        ],
        compiler_params=pltpu.CompilerParams(
            vmem_limit_bytes=100 * 1024 * 1024,
        ),
    )(Qr, Kt, Vt)

    return out


# device time: 33315 ns/iter; 9.1715x vs baseline; 1.1375x over previous
import jax
import jax.numpy as jnp
from jax import lax
from jax.experimental import pallas as pl
from jax.experimental.pallas import tpu as pltpu

SCALE = 64 ** -0.5
NSLOT = 4
W = 128


def kernel(Q, K, V):
    b, _, h, d = Q.shape
    kv = K.shape[1]
    bh = b // 2

    Qr = Q.reshape(b, h, d)
    Kt = jnp.transpose(K, (0, 2, 3, 1))
    Vt = jnp.transpose(V, (0, 2, 3, 1))

    def body(q_ref, k_hbm, v_hbm, out_ref,
             kbuf, vbuf, o_half, ox_oth, cbuf, cy_oth,
             ksems, vsems, xsend, xrecv, ysend, yrecv):
        my_x = lax.axis_index("x")
        my_y = lax.axis_index("y")
        b0 = my_y * bh
        ob0 = (1 - my_y) * bh
        peer_x = (1 - my_x, my_y)
        peer_y = (my_x, 1 - my_y)
        hh = h // 4

        def k_copy(bi, slot, half):
            sl = pl.ds(half * hh, hh)
            return pltpu.make_async_copy(
                k_hbm.at[b0 + bi, sl], kbuf.at[slot, sl], ksems.at[slot, half])

        def v_copy(bi, slot, half):
            sl = pl.ds(half * hh, hh)
            return pltpu.make_async_copy(
                v_hbm.at[b0 + bi, sl], vbuf.at[slot, sl], vsems.at[slot, half])

        barrier_sem = pltpu.get_barrier_semaphore()
        for nbr in (peer_x, peer_y):
            pl.semaphore_signal(
                barrier_sem, inc=1,
                device_id=nbr, device_id_type=pl.DeviceIdType.MESH,
            )
        pl.semaphore_wait(barrier_sem, 2)

        def x_rdma(bi):
            return pltpu.make_async_remote_copy(
                src_ref=o_half.at[pl.ds(bi, 1)],
                dst_ref=ox_oth.at[pl.ds(bi, 1)],
                send_sem=xsend.at[bi], recv_sem=xrecv.at[bi],
                device_id=peer_x, device_id_type=pl.DeviceIdType.MESH,
            )

        def y_rdma(r):
            sl = pl.ds(r, 1)
            return pltpu.make_async_remote_copy(
                src_ref=cbuf.at[sl], dst_ref=cy_oth.at[sl],
                send_sem=ysend.at[r], recv_sem=yrecv.at[r],
                device_id=peer_y, device_id_type=pl.DeviceIdType.MESH,
            )

        def combine_and_forward(r):
            x_rdma(r).wait_recv()
            sl = pl.ds(r, 1)
            cbuf[sl] = o_half[sl] + ox_oth[sl]
            y_rdma(r).start()

        for bi in range(NSLOT):
            for half in range(4):
                k_copy(bi, bi % NSLOT, half).start()
                v_copy(bi, bi % NSLOT, half).start()

        for bi in range(bh):
            slot = bi % NSLOT
            for half in range(4):
                k_copy(bi, slot, half).wait()
                v_copy(bi, slot, half).wait()

            q3 = q_ref[pl.ds(b0 + bi, 1)][0][:, None, :].astype(jnp.bfloat16)
            kt = kbuf[slot].astype(jnp.bfloat16)
            vt = vbuf[slot].astype(jnp.bfloat16)

            s3 = lax.dot_general(
                q3, kt,
                dimension_numbers=(((2,), (1,)), ((0,), (0,))),
                preferred_element_type=jnp.float32,
            )
            p3 = jnp.exp(s3 * SCALE)
            l3 = p3.sum(axis=2)
            o3 = lax.dot_general(
                vt, p3.astype(jnp.bfloat16),
                dimension_numbers=(((2,), (2,)), ((0,), (0,))),
                preferred_element_type=jnp.float32,
            )

            o_half[pl.ds(bi, 1), :, 0:d] = o3[:, :, 0][None]
            o_half[pl.ds(bi, 1), :, d:d + 1] = l3[None]

            x_rdma(bi).start()

            if bi >= 1:
                combine_and_forward(bi - 1)

            if bi + NSLOT < bh:
                for half in range(4):
                    k_copy(bi + NSLOT, slot, half).start()
                    v_copy(bi + NSLOT, slot, half).start()

        combine_and_forward(bh - 1)
        for bi in range(bh):
            x_rdma(bi).wait_send()

        mine = cbuf[...]
        out_ref[pl.ds(b0, bh)] = (mine[:, :, 0:d] / mine[:, :, d:d + 1])[:, None]

        for r in range(bh):
            y_rdma(r).wait()
        theirs = cy_oth[...]
        out_ref[pl.ds(ob0, bh)] = (theirs[:, :, 0:d] / theirs[:, :, d:d + 1])[:, None]

    out = pl.pallas_call(
        body,
        in_specs=[
            pl.BlockSpec(memory_space=pltpu.MemorySpace.VMEM),
            pl.BlockSpec(memory_space=pl.ANY),
            pl.BlockSpec(memory_space=pl.ANY),
        ],
        out_specs=pl.BlockSpec(memory_space=pltpu.MemorySpace.VMEM),
        out_shape=jax.ShapeDtypeStruct((b, 1, h, d), jnp.float32),
        scratch_shapes=[
            pltpu.VMEM((NSLOT, h, d, kv), jnp.float32),
            pltpu.VMEM((NSLOT, h, d, kv), jnp.float32),
            pltpu.VMEM((bh, h, W), jnp.float32),
            pltpu.VMEM((bh, h, W), jnp.float32),
            pltpu.VMEM((bh, h, W), jnp.float32),
            pltpu.VMEM((bh, h, W), jnp.float32),
            pltpu.SemaphoreType.DMA((NSLOT, 4)),
            pltpu.SemaphoreType.DMA((NSLOT, 4)),
            pltpu.SemaphoreType.DMA((bh,)),
            pltpu.SemaphoreType.DMA((bh,)),
            pltpu.SemaphoreType.DMA((bh,)),
            pltpu.SemaphoreType.DMA((bh,)),
        ],
        compiler_params=pltpu.CompilerParams(
            vmem_limit_bytes=100 * 1024 * 1024,
            collective_id=0,
        ),
    )(Qr, Kt, Vt)

    return out
